# baseline (speedup 1.0000x reference)
import functools

import jax
import jax.numpy as jnp
from jax import lax
from jax.experimental import pallas as pl
from jax.experimental.pallas import tpu as pltpu
from jax.experimental.pallas import tpu_sc as plsc

_NC, _NS, _L = 2, 16, 16
_NW = _NC * _NS
_B = 16384
_PER_W = _B // _NW
_C = 64
_NCHUNK = _PER_W // _C
_D = 64
_W = 128
_MARGIN = 1.0


def _rsqrt(x):
    i = lax.bitcast_convert_type(x, jnp.int32)
    i = jnp.int32(0x5F3759DF) - lax.shift_right_arithmetic(i, 1)
    y = lax.bitcast_convert_type(i, jnp.float32)
    for _ in range(3):
        y = y * (1.5 - 0.5 * x * y * y)
    return y


def _sc_body(sp, tp, sn, tn, r, node, link, norm, out,
             isp, itp, isn, itn, ir,
             bufs0, bufs1, accv, sem0, sem1):
    wid = lax.axis_index("s") * _NC + lax.axis_index("c")
    base = wid * _PER_W
    iota = lax.iota(jnp.int32, _L)
    bufs = (bufs0, bufs1)
    sems = (sem0, sem1)

    def load_idx(c):
        off = pl.multiple_of(base + c * _C, _C)
        pltpu.sync_copy(sp.at[pl.ds(off, _C)], isp.at[c])
        pltpu.sync_copy(tp.at[pl.ds(off, _C)], itp.at[c])
        pltpu.sync_copy(sn.at[pl.ds(off, _C)], isn.at[c])
        pltpu.sync_copy(tn.at[pl.ds(off, _C)], itn.at[c])
        pltpu.sync_copy(r.at[pl.ds(off, _C)], ir.at[c])

    def fire(c, k):
        sl = pl.ds(0, _W)
        return [
            pltpu.async_copy(node.at[isp.at[c], sl], bufs[k].at[0], sems[k]),
            pltpu.async_copy(node.at[itp.at[c], sl], bufs[k].at[1], sems[k]),
            pltpu.async_copy(node.at[isn.at[c], sl], bufs[k].at[2], sems[k]),
            pltpu.async_copy(node.at[itn.at[c], sl], bufs[k].at[3], sems[k]),
            pltpu.async_copy(link.at[ir.at[c], sl], bufs[k].at[4], sems[k]),
            pltpu.async_copy(norm.at[ir.at[c], sl], bufs[k].at[5], sems[k]),
        ]

    load_idx(0)
    pend = fire(0, 0)

    acc = jnp.zeros((_L,), jnp.float32)
    for c in range(_NCHUNK):
        k = c % 2
        if c + 1 < _NCHUNK:
            load_idx(c + 1)
            nxt = fire(c + 1, 1 - k)
        else:
            nxt = None
        for cp in pend:
            cp.wait()
        pend = nxt
        bsp, btp, bsn, btn, brm, bw = (bufs[k].at[i] for i in range(6))

        def gbody(g, acc):
            z6 = (jnp.zeros((_L,), jnp.float32),) * 6

            @plsc.parallel_loop(0, _L, 1, unroll=2, carry=z6)
            def ebody(l, carry):
                qp, mp, qn, mn, ww, rw = carry
                e = g * _L + l
                z = jnp.zeros((_L,), jnp.float32)
                vqp, vmp, vqn, vmn, vww, vrw = z, z, z, z, z, z
                for kk in range(_D // _L):
                    ksl = pl.ds(kk * _L, _L)
                    vsp = bsp[e, ksl]
                    vtp = btp[e, ksl]
                    vsn = bsn[e, ksl]
                    vtn = btn[e, ksl]
                    vr = brm[e, ksl]
                    vw = bw[e, ksl]
                    bpv = vsp - vtp + vr
                    bnv = vsn - vtn + vr
                    vqp = vqp + bpv * bpv
                    vmp = vmp + bpv * vw
                    vqn = vqn + bnv * bnv
                    vmn = vmn + bnv * vw
                    vww = vww + vw * vw
                    vrw = vrw + vr * vw
                sel = iota == l
                qp = qp + jnp.where(sel, jnp.sum(vqp), 0.0)
                mp = mp + jnp.where(sel, jnp.sum(vmp), 0.0)
                qn = qn + jnp.where(sel, jnp.sum(vqn), 0.0)
                mn = mn + jnp.where(sel, jnp.sum(vmn), 0.0)
                ww = ww + jnp.where(sel, jnp.sum(vww), 0.0)
                rw = rw + jnp.where(sel, jnp.sum(vrw), 0.0)
                return qp, mp, qn, mn, ww, rw

            qp, mp, qn, mn, ww, rw = ebody
            cfp = (mp - rw) / ww
            cfn = (mn - rw) / ww
            ddp = qp - 2.0 * cfp * mp + cfp * cfp * ww
            ddn = qn - 2.0 * cfn * mn + cfn * cfn * ww
            ddp = jnp.maximum(ddp, 1e-20)
            ddn = jnp.maximum(ddn, 1e-20)
            dp = ddp * _rsqrt(ddp)
            dn = ddn * _rsqrt(ddn)
            return acc + jnp.maximum(0.0, dp - dn + _MARGIN)

        acc = lax.fori_loop(0, _C // _L, gbody, acc)
    accv[...] = acc
    pltpu.sync_copy(accv, out.at[pl.ds(wid * _L, _L)])


_mesh = plsc.VectorSubcoreMesh(core_axis_name="c", subcore_axis_name="s")

_sc_kernel = pl.kernel(
    _sc_body,
    out_type=jax.ShapeDtypeStruct((_NW * _L,), jnp.float32),
    mesh=_mesh,
    compiler_params=pltpu.CompilerParams(
        needs_layout_passes=False, use_tc_tiling_on_sc=True),
    scratch_types=[
        pltpu.VMEM((_NCHUNK, _C), jnp.int32),
        pltpu.VMEM((_NCHUNK, _C), jnp.int32),
        pltpu.VMEM((_NCHUNK, _C), jnp.int32),
        pltpu.VMEM((_NCHUNK, _C), jnp.int32),
        pltpu.VMEM((_NCHUNK, _C), jnp.int32),
        pltpu.VMEM((6, _C, _W), jnp.float32),
        pltpu.VMEM((6, _C, _W), jnp.float32),
        pltpu.VMEM((_L,), jnp.float32),
        pltpu.SemaphoreType.DMA,
        pltpu.SemaphoreType.DMA,
    ],
)


def kernel(sp, tp, sn, tn, r, node_emb, link_emb, norm_vector):
    sp = sp.astype(jnp.int32)
    tp = tp.astype(jnp.int32)
    sn = sn.astype(jnp.int32)
    tn = tn.astype(jnp.int32)
    r = r.astype(jnp.int32)
    partial = _sc_kernel(sp, tp, sn, tn, r, node_emb, link_emb, norm_vector)
    return jnp.sum(partial) / _B


# speedup vs baseline: 1.0061x; 1.0061x over previous
import functools

import jax
import jax.numpy as jnp
from jax import lax
from jax.experimental import pallas as pl
from jax.experimental.pallas import tpu as pltpu
from jax.experimental.pallas import tpu_sc as plsc

_NC, _NS, _L = 2, 16, 16
_NW = _NC * _NS
_B = 16384
_PER_W = _B // _NW
_C = 64
_NCHUNK = _PER_W // _C
_D = 64
_W = 128
_MARGIN = 1.0


def _rsqrt(x):
    i = lax.bitcast_convert_type(x, jnp.int32)
    i = jnp.int32(0x5F3759DF) - lax.shift_right_arithmetic(i, 1)
    y = lax.bitcast_convert_type(i, jnp.float32)
    for _ in range(3):
        y = y * (1.5 - 0.5 * x * y * y)
    return y


def _sc_body(sp, tp, sn, tn, r, node, link, norm, out,
             isp, itp, isn, itn, ir,
             bufs0, bufs1, accv, sem0, sem1):
    wid = lax.axis_index("s") * _NC + lax.axis_index("c")
    base = wid * _PER_W
    iota = lax.iota(jnp.int32, _L)
    bufs = (bufs0, bufs1)
    sems = (sem0, sem1)

    def load_idx(c):
        off = pl.multiple_of(base + c * _C, _C)
        pltpu.sync_copy(sp.at[pl.ds(off, _C)], isp.at[c])
        pltpu.sync_copy(tp.at[pl.ds(off, _C)], itp.at[c])
        pltpu.sync_copy(sn.at[pl.ds(off, _C)], isn.at[c])
        pltpu.sync_copy(tn.at[pl.ds(off, _C)], itn.at[c])
        pltpu.sync_copy(r.at[pl.ds(off, _C)], ir.at[c])

    def fire(c, k):
        sl = pl.ds(0, _W)
        return [
            pltpu.async_copy(node.at[isp.at[c], sl], bufs[k].at[0], sems[k]),
            pltpu.async_copy(node.at[itp.at[c], sl], bufs[k].at[1], sems[k]),
            pltpu.async_copy(node.at[isn.at[c], sl], bufs[k].at[2], sems[k]),
            pltpu.async_copy(node.at[itn.at[c], sl], bufs[k].at[3], sems[k]),
            pltpu.async_copy(link.at[ir.at[c], sl], bufs[k].at[4], sems[k]),
            pltpu.async_copy(norm.at[ir.at[c], sl], bufs[k].at[5], sems[k]),
        ]

    load_idx(0)
    pend = fire(0, 0)

    acc = jnp.zeros((_L,), jnp.float32)
    for c in range(_NCHUNK):
        k = c % 2
        if c + 1 < _NCHUNK:
            load_idx(c + 1)
            nxt = fire(c + 1, 1 - k)
        else:
            nxt = None
        for cp in pend:
            cp.wait()
        pend = nxt
        bsp, btp, bsn, btn, brm, bw = (bufs[k].at[i] for i in range(6))

        def gbody(g, acc):
            def ebody(l, carry):
                qp, mp, qn, mn, ww, rw = carry
                e = g * _L + l
                z = jnp.zeros((_L,), jnp.float32)
                vqp, vmp, vqn, vmn, vww, vrw = z, z, z, z, z, z
                for kk in range(_D // _L):
                    ksl = pl.ds(kk * _L, _L)
                    vsp = bsp[e, ksl]
                    vtp = btp[e, ksl]
                    vsn = bsn[e, ksl]
                    vtn = btn[e, ksl]
                    vr = brm[e, ksl]
                    vw = bw[e, ksl]
                    bpv = vsp - vtp + vr
                    bnv = vsn - vtn + vr
                    vqp = vqp + bpv * bpv
                    vmp = vmp + bpv * vw
                    vqn = vqn + bnv * bnv
                    vmn = vmn + bnv * vw
                    vww = vww + vw * vw
                    vrw = vrw + vr * vw
                sel = iota == l
                qp = qp + jnp.where(sel, jnp.sum(vqp), 0.0)
                mp = mp + jnp.where(sel, jnp.sum(vmp), 0.0)
                qn = qn + jnp.where(sel, jnp.sum(vqn), 0.0)
                mn = mn + jnp.where(sel, jnp.sum(vmn), 0.0)
                ww = ww + jnp.where(sel, jnp.sum(vww), 0.0)
                rw = rw + jnp.where(sel, jnp.sum(vrw), 0.0)
                return qp, mp, qn, mn, ww, rw

            z = jnp.zeros((_L,), jnp.float32)
            qp, mp, qn, mn, ww, rw = lax.fori_loop(
                0, _L, ebody, (z, z, z, z, z, z))
            cfp = (mp - rw) / ww
            cfn = (mn - rw) / ww
            ddp = qp - 2.0 * cfp * mp + cfp * cfp * ww
            ddn = qn - 2.0 * cfn * mn + cfn * cfn * ww
            ddp = jnp.maximum(ddp, 1e-20)
            ddn = jnp.maximum(ddn, 1e-20)
            dp = ddp * _rsqrt(ddp)
            dn = ddn * _rsqrt(ddn)
            return acc + jnp.maximum(0.0, dp - dn + _MARGIN)

        acc = lax.fori_loop(0, _C // _L, gbody, acc)
    accv[...] = acc
    pltpu.sync_copy(accv, out.at[pl.ds(wid * _L, _L)])


_mesh = plsc.VectorSubcoreMesh(core_axis_name="c", subcore_axis_name="s")

_sc_kernel = pl.kernel(
    _sc_body,
    out_type=jax.ShapeDtypeStruct((_NW * _L,), jnp.float32),
    mesh=_mesh,
    compiler_params=pltpu.CompilerParams(
        needs_layout_passes=False, use_tc_tiling_on_sc=True),
    scratch_types=[
        pltpu.VMEM((_NCHUNK, _C), jnp.int32),
        pltpu.VMEM((_NCHUNK, _C), jnp.int32),
        pltpu.VMEM((_NCHUNK, _C), jnp.int32),
        pltpu.VMEM((_NCHUNK, _C), jnp.int32),
        pltpu.VMEM((_NCHUNK, _C), jnp.int32),
        pltpu.VMEM((6, _C, _W), jnp.float32),
        pltpu.VMEM((6, _C, _W), jnp.float32),
        pltpu.VMEM((_L,), jnp.float32),
        pltpu.SemaphoreType.DMA,
        pltpu.SemaphoreType.DMA,
    ],
)


def kernel(sp, tp, sn, tn, r, node_emb, link_emb, norm_vector):
    sp = sp.astype(jnp.int32)
    tp = tp.astype(jnp.int32)
    sn = sn.astype(jnp.int32)
    tn = tn.astype(jnp.int32)
    r = r.astype(jnp.int32)
    partial = _sc_kernel(sp, tp, sn, tn, r, node_emb, link_emb, norm_vector)
    return jnp.sum(partial) / _B
